# baseline (speedup 1.0000x reference)
import jax
import jax.numpy as jnp
from jax.experimental import pallas as pl
from jax.experimental.pallas import tpu as pltpu

_B = 64
_H = 128
_PAD = 8


def _rot_step(y, d):
    fill = jnp.full((y.shape[0], d, y.shape[2]), -jnp.inf, y.dtype)
    return jnp.maximum(y, jnp.concatenate([y[:, d:, :], fill], axis=1))


def _reg_phase(y):
    for d in (1, 8, 16, 32, 64):
        y = _rot_step(y, d)
    return y


def _corner_pool_kernel(x_ref, o_ref, s0_ref, t0_ref, s1_ref, t1_ref):
    @pl.when(pl.program_id(0) == 0)
    def _():
        pad = jnp.full((_B // 2, _PAD, _H), -jnp.inf, jnp.float32)
        s0_ref[:, _H:, :] = pad
        t0_ref[:, _H:, :] = pad
        s1_ref[:, _H:, :] = pad
        t1_ref[:, _H:, :] = pad

    hb = _B // 2
    halves = ((slice(0, hb), s0_ref, t0_ref), (slice(hb, _B), s1_ref, t1_ref))

    for b, s_ref, t_ref in halves:
        s_ref[:, :_H, :] = _reg_phase(x_ref[b])
    for b, s_ref, t_ref in halves:
        t_ref[:, :_H, :] = jnp.maximum(s_ref[:, :_H, :], s_ref[:, 2 : _H + 2, :])
    for b, s_ref, t_ref in halves:
        y = jnp.maximum(t_ref[:, :_H, :], t_ref[:, 4 : _H + 4, :])
        s_ref[:, :_H, :] = _reg_phase(jnp.swapaxes(y, 1, 2))
    for b, s_ref, t_ref in halves:
        t_ref[:, :_H, :] = jnp.maximum(s_ref[:, :_H, :], s_ref[:, 2 : _H + 2, :])
    for b, s_ref, t_ref in halves:
        z = jnp.maximum(t_ref[:, :_H, :], t_ref[:, 4 : _H + 4, :])
        o_ref[b] = jnp.swapaxes(z + z, 1, 2)


@jax.jit
def kernel(x):
    N, C, H, W = x.shape
    xr = x.reshape(N * C, H, W)
    grid = (N * C // _B,)
    out = pl.pallas_call(
        _corner_pool_kernel,
        grid=grid,
        in_specs=[pl.BlockSpec((_B, H, W), lambda i: (i, 0, 0))],
        out_specs=pl.BlockSpec((_B, H, W), lambda i: (i, 0, 0)),
        out_shape=jax.ShapeDtypeStruct((N * C, H, W), x.dtype),
        scratch_shapes=[
            pltpu.VMEM((_B // 2, H + _PAD, W), jnp.float32),
            pltpu.VMEM((_B // 2, H + _PAD, W), jnp.float32),
            pltpu.VMEM((_B // 2, H + _PAD, W), jnp.float32),
            pltpu.VMEM((_B // 2, H + _PAD, W), jnp.float32),
        ],
        compiler_params=pltpu.CompilerParams(
            dimension_semantics=("parallel",),
        ),
    )(xr)
    return out.reshape(N, C, H, W)


# speedup vs baseline: 1.0094x; 1.0094x over previous
import jax
import jax.numpy as jnp
from jax.experimental import pallas as pl
from jax.experimental.pallas import tpu as pltpu

_B = 128
_H = 128
_PAD = 8


def _rot_step(y, d):
    fill = jnp.full((y.shape[0], d, y.shape[2]), -jnp.inf, y.dtype)
    return jnp.maximum(y, jnp.concatenate([y[:, d:, :], fill], axis=1))


def _reg_phase(y):
    for d in (1, 8, 16, 32, 64):
        y = _rot_step(y, d)
    return y


def _corner_pool_kernel(x_ref, o_ref, s0_ref, t0_ref, s1_ref, t1_ref):
    @pl.when(pl.program_id(0) == 0)
    def _():
        pad = jnp.full((_B // 2, _PAD, _H), -jnp.inf, jnp.float32)
        s0_ref[:, _H:, :] = pad
        t0_ref[:, _H:, :] = pad
        s1_ref[:, _H:, :] = pad
        t1_ref[:, _H:, :] = pad

    hb = _B // 2
    halves = ((slice(0, hb), s0_ref, t0_ref), (slice(hb, _B), s1_ref, t1_ref))

    for b, s_ref, t_ref in halves:
        s_ref[:, :_H, :] = _reg_phase(x_ref[b])
    for b, s_ref, t_ref in halves:
        t_ref[:, :_H, :] = jnp.maximum(s_ref[:, :_H, :], s_ref[:, 2 : _H + 2, :])
    for b, s_ref, t_ref in halves:
        y = jnp.maximum(t_ref[:, :_H, :], t_ref[:, 4 : _H + 4, :])
        s_ref[:, :_H, :] = _reg_phase(jnp.swapaxes(y, 1, 2))
    for b, s_ref, t_ref in halves:
        t_ref[:, :_H, :] = jnp.maximum(s_ref[:, :_H, :], s_ref[:, 2 : _H + 2, :])
    for b, s_ref, t_ref in halves:
        z = jnp.maximum(t_ref[:, :_H, :], t_ref[:, 4 : _H + 4, :])
        o_ref[b] = jnp.swapaxes(z + z, 1, 2)


@jax.jit
def kernel(x):
    N, C, H, W = x.shape
    xr = x.reshape(N * C, H, W)
    grid = (N * C // _B,)
    out = pl.pallas_call(
        _corner_pool_kernel,
        grid=grid,
        in_specs=[pl.BlockSpec((_B, H, W), lambda i: (i, 0, 0))],
        out_specs=pl.BlockSpec((_B, H, W), lambda i: (i, 0, 0)),
        out_shape=jax.ShapeDtypeStruct((N * C, H, W), x.dtype),
        scratch_shapes=[
            pltpu.VMEM((_B // 2, H + _PAD, W), jnp.float32),
            pltpu.VMEM((_B // 2, H + _PAD, W), jnp.float32),
            pltpu.VMEM((_B // 2, H + _PAD, W), jnp.float32),
            pltpu.VMEM((_B // 2, H + _PAD, W), jnp.float32),
        ],
        compiler_params=pltpu.CompilerParams(
            dimension_semantics=("arbitrary",),
            vmem_limit_bytes=56 * 1024 * 1024,
        ),
    )(xr)
    return out.reshape(N, C, H, W)


# speedup vs baseline: 1.0433x; 1.0336x over previous
import jax
import jax.numpy as jnp
from jax.experimental import pallas as pl
from jax.experimental.pallas import tpu as pltpu

_B = 128


def _sublane_suffix_max(y):
    neg = jnp.float32(-jnp.inf)
    d = 1
    while d < y.shape[1]:
        fill = jnp.full((y.shape[0], d, y.shape[2]), neg, y.dtype)
        y = jnp.maximum(y, jnp.concatenate([y[:, d:, :], fill], axis=1))
        d *= 2
    return y


def _corner_pool_kernel(x_ref, o_ref):
    y = _sublane_suffix_max(x_ref[...])
    y = jnp.swapaxes(y, 1, 2)
    y = _sublane_suffix_max(y)
    o_ref[...] = jnp.swapaxes(y + y, 1, 2)


@jax.jit
def kernel(x):
    N, C, H, W = x.shape
    xr = x.reshape(N * C, H, W)
    grid = (N * C // _B,)
    out = pl.pallas_call(
        _corner_pool_kernel,
        grid=grid,
        in_specs=[pl.BlockSpec((_B, H, W), lambda i: (i, 0, 0))],
        out_specs=pl.BlockSpec((_B, H, W), lambda i: (i, 0, 0)),
        out_shape=jax.ShapeDtypeStruct((N * C, H, W), x.dtype),
        compiler_params=pltpu.CompilerParams(
            dimension_semantics=("parallel",),
        ),
    )(xr)
    return out.reshape(N, C, H, W)
